# Initial kernel scaffold; baseline (speedup 1.0000x reference)
import jax
import jax.numpy as jnp
from jax.experimental import pallas as pl


def kernel(tokens, states, router_w, router_b, q_w, q_b, k_w, k_b, v_w, v_b, out_w, out_b):
    raise NotImplementedError("write your pallas kernel here")


# speedup vs baseline: 5.0228x; 5.0228x over previous
import jax
import jax.numpy as jnp
from jax import lax
from jax.experimental import pallas as pl
from jax.experimental.pallas import tpu as pltpu

B, S, N = 2, 2048, 1024
TD, SD, H, K = 1024, 1024, 16, 8
HD = SD // H
SCALE = HD ** -0.5
BS = 256


def _kv_body(states_ref, kwT_ref, vwT_ref, kb_ref, vb_ref, kst_ref, vst_ref):
    st = states_ref[0]
    kst_ref[0] = jnp.dot(st, kwT_ref[...], preferred_element_type=jnp.float32) + kb_ref[...]
    vst_ref[0] = jnp.dot(st, vwT_ref[...], preferred_element_type=jnp.float32) + vb_ref[...]


def _attn_body(tok_ref, statesT_ref, rwT_ref, rb_ref, qwT_ref, qb_ref,
               kst_ref, vst_ref, owT_ref, ob_ref, out_ref, attn_ref):
    tok = tok_ref[0]
    routed = jnp.dot(tok, rwT_ref[...], preferred_element_type=jnp.float32) + rb_ref[...]
    scores = jnp.dot(routed, statesT_ref[0], preferred_element_type=jnp.float32)

    iota_n = lax.broadcasted_iota(jnp.int32, (BS, N), 1)
    x = scores
    idxs = []
    for _ in range(K):
        m = jnp.max(x, axis=-1, keepdims=True)
        am = jnp.min(jnp.where(x == m, iota_n, N), axis=-1, keepdims=True)
        idxs.append(am)
        x = jnp.where(iota_n == am, -3e38, x)

    q = jnp.dot(tok, qwT_ref[...], preferred_element_type=jnp.float32) + qb_ref[...]

    d_iota = lax.broadcasted_iota(jnp.int32, (SD, H), 0)
    h_iota = lax.broadcasted_iota(jnp.int32, (SD, H), 1)
    dmat = (d_iota // HD == h_iota).astype(jnp.float32)
    h2 = lax.broadcasted_iota(jnp.int32, (H, SD), 0)
    d2 = lax.broadcasted_iota(jnp.int32, (H, SD), 1)
    dmat2 = (d2 // HD == h2).astype(jnp.float32)

    logits = []
    for kk in range(K):
        onehot = (iota_n == idxs[kk]).astype(jnp.float32)
        ksel = jnp.dot(onehot, kst_ref[0], preferred_element_type=jnp.float32)
        logits.append(jnp.dot(ksel * q, dmat, preferred_element_type=jnp.float32) * SCALE)

    mx = logits[0]
    for kk in range(1, K):
        mx = jnp.maximum(mx, logits[kk])
    es = [jnp.exp(l - mx) for l in logits]
    denom = es[0]
    for kk in range(1, K):
        denom = denom + es[kk]

    o = jnp.zeros((BS, SD), jnp.float32)
    for kk in range(K):
        w = es[kk] / denom
        attn_ref[0, kk] = w
        onehot = (iota_n == idxs[kk]).astype(jnp.float32)
        vsel = jnp.dot(onehot, vst_ref[0], preferred_element_type=jnp.float32)
        wexp = jnp.dot(w, dmat2, preferred_element_type=jnp.float32)
        o = o + wexp * vsel

    out_ref[0] = jnp.dot(o, owT_ref[...], preferred_element_type=jnp.float32) + ob_ref[...]


def kernel(tokens, states, router_w, router_b, q_w, q_b, k_w, k_b, v_w, v_b, out_w, out_b):
    statesT = states.transpose(0, 2, 1)
    rwT, qwT, kwT, vwT, owT = router_w.T, q_w.T, k_w.T, v_w.T, out_w.T
    rb = router_b.reshape(1, SD)
    qb = q_b.reshape(1, SD)
    kb = k_b.reshape(1, SD)
    vb = v_b.reshape(1, SD)
    ob = out_b.reshape(1, SD)

    kst, vst = pl.pallas_call(
        _kv_body,
        grid=(B,),
        in_specs=[
            pl.BlockSpec((1, N, SD), lambda b: (b, 0, 0)),
            pl.BlockSpec((SD, SD), lambda b: (0, 0)),
            pl.BlockSpec((SD, SD), lambda b: (0, 0)),
            pl.BlockSpec((1, SD), lambda b: (0, 0)),
            pl.BlockSpec((1, SD), lambda b: (0, 0)),
        ],
        out_specs=[
            pl.BlockSpec((1, N, SD), lambda b: (b, 0, 0)),
            pl.BlockSpec((1, N, SD), lambda b: (b, 0, 0)),
        ],
        out_shape=[
            jax.ShapeDtypeStruct((B, N, SD), jnp.float32),
            jax.ShapeDtypeStruct((B, N, SD), jnp.float32),
        ],
    )(states, kwT, vwT, kb, vb)

    out, attn = pl.pallas_call(
        _attn_body,
        grid=(B, S // BS),
        in_specs=[
            pl.BlockSpec((1, BS, TD), lambda b, s: (b, s, 0)),
            pl.BlockSpec((1, SD, N), lambda b, s: (b, 0, 0)),
            pl.BlockSpec((TD, SD), lambda b, s: (0, 0)),
            pl.BlockSpec((1, SD), lambda b, s: (0, 0)),
            pl.BlockSpec((TD, SD), lambda b, s: (0, 0)),
            pl.BlockSpec((1, SD), lambda b, s: (0, 0)),
            pl.BlockSpec((1, N, SD), lambda b, s: (b, 0, 0)),
            pl.BlockSpec((1, N, SD), lambda b, s: (b, 0, 0)),
            pl.BlockSpec((SD, SD), lambda b, s: (0, 0)),
            pl.BlockSpec((1, SD), lambda b, s: (0, 0)),
        ],
        out_specs=[
            pl.BlockSpec((1, BS, SD), lambda b, s: (b, s, 0)),
            pl.BlockSpec((1, K, BS, H), lambda b, s: (b, 0, s, 0)),
        ],
        out_shape=[
            jax.ShapeDtypeStruct((B, S, SD), jnp.float32),
            jax.ShapeDtypeStruct((B, K, S, H), jnp.float32),
        ],
    )(tokens, statesT, rwT, rb, qwT, qb, kst, vst, owT, ob)

    return (out, attn.transpose(0, 3, 2, 1))


# speedup vs baseline: 5.1511x; 1.0255x over previous
import jax
import jax.numpy as jnp
from jax import lax
from jax.experimental import pallas as pl
from jax.experimental.pallas import tpu as pltpu

B, S, N = 2, 2048, 1024
TD, SD, H, K = 1024, 1024, 16, 8
HD = SD // H
SCALE = HD ** -0.5
BS = 256


def _kv_body(states_ref, kwT_ref, vwT_ref, kb_ref, vb_ref, kst_ref, vst_ref):
    st = states_ref[0]
    kst_ref[0] = jnp.dot(st, kwT_ref[...], preferred_element_type=jnp.float32) + kb_ref[...]
    vst_ref[0] = jnp.dot(st, vwT_ref[...], preferred_element_type=jnp.float32) + vb_ref[...]


def _attn_body(tok_ref, statesT_ref, rwT_ref, rb_ref, qwT_ref, qb_ref,
               kst_ref, vst_ref, owT_ref, ob_ref, out_ref, attn_ref):
    tok = tok_ref[0]
    routed = jnp.dot(tok, rwT_ref[...], preferred_element_type=jnp.float32) + rb_ref[...]
    scores = jnp.dot(routed, statesT_ref[0], preferred_element_type=jnp.float32)

    iota_n = lax.broadcasted_iota(jnp.int32, (BS, N), 1)
    x = scores
    idxs = []
    for _ in range(K):
        m = jnp.max(x, axis=-1, keepdims=True)
        am = jnp.min(jnp.where(x == m, iota_n, N), axis=-1, keepdims=True)
        idxs.append(am)
        x = jnp.where(iota_n == am, -3e38, x)

    q = jnp.dot(tok.astype(jnp.bfloat16), qwT_ref[...].astype(jnp.bfloat16),
                preferred_element_type=jnp.float32) + qb_ref[...]

    d_iota = lax.broadcasted_iota(jnp.int32, (SD, H), 0)
    h_iota = lax.broadcasted_iota(jnp.int32, (SD, H), 1)
    dmat = (d_iota // HD == h_iota).astype(jnp.float32)
    h2 = lax.broadcasted_iota(jnp.int32, (H, SD), 0)
    d2 = lax.broadcasted_iota(jnp.int32, (H, SD), 1)
    dmat2 = (d2 // HD == h2).astype(jnp.float32)

    kst_b = kst_ref[0].astype(jnp.bfloat16)
    vst_b = vst_ref[0].astype(jnp.bfloat16)
    logits = []
    for kk in range(K):
        onehot = (iota_n == idxs[kk]).astype(jnp.bfloat16)
        ksel = jnp.dot(onehot, kst_b, preferred_element_type=jnp.float32)
        logits.append(jnp.dot(ksel * q, dmat, preferred_element_type=jnp.float32) * SCALE)

    mx = logits[0]
    for kk in range(1, K):
        mx = jnp.maximum(mx, logits[kk])
    es = [jnp.exp(l - mx) for l in logits]
    denom = es[0]
    for kk in range(1, K):
        denom = denom + es[kk]

    o = jnp.zeros((BS, SD), jnp.float32)
    for kk in range(K):
        w = es[kk] / denom
        attn_ref[0, kk] = w
        onehot = (iota_n == idxs[kk]).astype(jnp.bfloat16)
        vsel = jnp.dot(onehot, vst_b, preferred_element_type=jnp.float32)
        wexp = jnp.dot(w, dmat2, preferred_element_type=jnp.float32)
        o = o + wexp * vsel

    out_ref[0] = jnp.dot(o.astype(jnp.bfloat16), owT_ref[...].astype(jnp.bfloat16),
                         preferred_element_type=jnp.float32) + ob_ref[...]


def kernel(tokens, states, router_w, router_b, q_w, q_b, k_w, k_b, v_w, v_b, out_w, out_b):
    statesT = states.transpose(0, 2, 1)
    rwT, qwT, kwT, vwT, owT = router_w.T, q_w.T, k_w.T, v_w.T, out_w.T
    rb = router_b.reshape(1, SD)
    qb = q_b.reshape(1, SD)
    kb = k_b.reshape(1, SD)
    vb = v_b.reshape(1, SD)
    ob = out_b.reshape(1, SD)

    kst, vst = pl.pallas_call(
        _kv_body,
        grid=(B,),
        in_specs=[
            pl.BlockSpec((1, N, SD), lambda b: (b, 0, 0)),
            pl.BlockSpec((SD, SD), lambda b: (0, 0)),
            pl.BlockSpec((SD, SD), lambda b: (0, 0)),
            pl.BlockSpec((1, SD), lambda b: (0, 0)),
            pl.BlockSpec((1, SD), lambda b: (0, 0)),
        ],
        out_specs=[
            pl.BlockSpec((1, N, SD), lambda b: (b, 0, 0)),
            pl.BlockSpec((1, N, SD), lambda b: (b, 0, 0)),
        ],
        out_shape=[
            jax.ShapeDtypeStruct((B, N, SD), jnp.float32),
            jax.ShapeDtypeStruct((B, N, SD), jnp.float32),
        ],
    )(states, kwT, vwT, kb, vb)

    out, attn = pl.pallas_call(
        _attn_body,
        grid=(B, S // BS),
        in_specs=[
            pl.BlockSpec((1, BS, TD), lambda b, s: (b, s, 0)),
            pl.BlockSpec((1, SD, N), lambda b, s: (b, 0, 0)),
            pl.BlockSpec((TD, SD), lambda b, s: (0, 0)),
            pl.BlockSpec((1, SD), lambda b, s: (0, 0)),
            pl.BlockSpec((TD, SD), lambda b, s: (0, 0)),
            pl.BlockSpec((1, SD), lambda b, s: (0, 0)),
            pl.BlockSpec((1, N, SD), lambda b, s: (b, 0, 0)),
            pl.BlockSpec((1, N, SD), lambda b, s: (b, 0, 0)),
            pl.BlockSpec((SD, SD), lambda b, s: (0, 0)),
            pl.BlockSpec((1, SD), lambda b, s: (0, 0)),
        ],
        out_specs=[
            pl.BlockSpec((1, BS, SD), lambda b, s: (b, s, 0)),
            pl.BlockSpec((1, K, BS, H), lambda b, s: (b, 0, s, 0)),
        ],
        out_shape=[
            jax.ShapeDtypeStruct((B, S, SD), jnp.float32),
            jax.ShapeDtypeStruct((B, K, S, H), jnp.float32),
        ],
    )(tokens, statesT, rwT, rb, qwT, qb, kst, vst, owT, ob)

    return (out, attn.transpose(0, 3, 2, 1))
